# baseline (speedup 1.0000x reference)
import functools
import math

import jax
import jax.numpy as jnp
from jax import lax
from jax.experimental import pallas as pl
from jax.experimental.pallas import tpu as pltpu
from jax.experimental.pallas import tpu_sc as plsc

_NUM_CLASSES = 19
_THRESH = -math.log(0.7)
_N_MIN = 131072
_N_PIX = 8 * 512 * 512
_R = 256
_GB = 512 // _R
_GRID = 8 * _GB

_NW = 32
_PER_W = _N_PIX // _NW
_UNROLL = 8
_VREGS_PER_W = _PER_W // 16


def _bincount_body(lbl_hbm, out_hbm, lbl_v, tab_v):
    wid = lax.axis_index("s") * 2 + lax.axis_index("c")
    base = wid * _PER_W
    for c in range(_NUM_CLASSES):
        tab_v[pl.ds(c * 16, 16)] = jnp.zeros((16,), jnp.int32)
    pltpu.sync_copy(lbl_hbm.at[pl.ds(base, _PER_W)], lbl_v)
    lane = lax.iota(jnp.int32, 16)
    ones = jnp.ones((16,), jnp.int32)

    def body(i, carry):
        for u in range(_UNROLL):
            v = lbl_v[pl.ds((i * _UNROLL + u) * 16, 16)]
            plsc.addupdate_scatter(tab_v, [v * 16 + lane], ones)
        return carry

    lax.fori_loop(0, _VREGS_PER_W // _UNROLL, body, 0)
    pltpu.sync_copy(tab_v, out_hbm.at[wid])


@functools.lru_cache(maxsize=None)
def _bincount_call():
    return pl.kernel(
        _bincount_body,
        mesh=plsc.VectorSubcoreMesh(core_axis_name="c", subcore_axis_name="s"),
        out_type=jax.ShapeDtypeStruct((_NW, _NUM_CLASSES * 16), jnp.int32),
        scratch_types=[
            pltpu.VMEM((_PER_W,), jnp.int32),
            pltpu.VMEM((_NUM_CLASSES * 16,), jnp.int32),
        ],
        compiler_params=pltpu.CompilerParams(needs_layout_passes=False),
    )


def _loss_core(w_ref, x_ref, lbl_ref):
    x = x_ref[0]
    lbl = lbl_ref[0]
    m = x[0]
    for c in range(1, _NUM_CLASSES):
        m = jnp.maximum(m, x[c])
    s = jnp.exp(x[0] - m)
    for c in range(1, _NUM_CLASSES):
        s = s + jnp.exp(x[c] - m)
    lse = jnp.log(s) + m
    acc_x = x[0]
    acc_w = jnp.full(m.shape, w_ref[0], jnp.float32)
    for c in range(1, _NUM_CLASSES):
        sel = lbl == c
        acc_x = jnp.where(sel, x[c], acc_x)
        acc_w = jnp.where(sel, w_ref[c], acc_w)
    return acc_w * (lse - acc_x)


def _loss_stats_body(w_ref, x_ref, lbl_ref, sum_ref, cnt_ref):
    loss = _loss_core(w_ref, x_ref, lbl_ref)
    mask = loss > _THRESH
    sum_ref[0] = jnp.full((8, 128), jnp.sum(jnp.where(mask, loss, 0.0)),
                          jnp.float32)
    cnt_ref[0] = jnp.full((8, 128), jnp.sum(mask.astype(jnp.float32)),
                          jnp.float32)


def _loss_values_body(w_ref, x_ref, lbl_ref, loss_ref):
    loss_ref[0] = _loss_core(w_ref, x_ref, lbl_ref)


_dense_in_specs = [
    pl.BlockSpec(memory_space=pltpu.SMEM),
    pl.BlockSpec((1, _NUM_CLASSES, _R, 512), lambda i: (i // _GB, 0, i % _GB, 0)),
    pl.BlockSpec((1, _R, 512), lambda i: (i // _GB, i % _GB, 0)),
]

_loss_stats_call = pl.pallas_call(
    _loss_stats_body,
    grid=(_GRID,),
    in_specs=_dense_in_specs,
    out_specs=[
        pl.BlockSpec((1, 8, 128), lambda i: (i, 0, 0)),
        pl.BlockSpec((1, 8, 128), lambda i: (i, 0, 0)),
    ],
    out_shape=[
        jax.ShapeDtypeStruct((_GRID, 8, 128), jnp.float32),
        jax.ShapeDtypeStruct((_GRID, 8, 128), jnp.float32),
    ],
    compiler_params=pltpu.CompilerParams(dimension_semantics=("parallel",)),
)

_loss_values_call = pl.pallas_call(
    _loss_values_body,
    grid=(_GRID,),
    in_specs=_dense_in_specs,
    out_specs=pl.BlockSpec((1, _R, 512), lambda i: (i // _GB, i % _GB, 0)),
    out_shape=jax.ShapeDtypeStruct((8, 512, 512), jnp.float32),
    compiler_params=pltpu.CompilerParams(dimension_semantics=("parallel",)),
)


def _topk_sum_body(x_ref, out_ref):
    x = x_ref[...]
    bits = lax.bitcast_convert_type(x, jnp.int32)
    k = jnp.float32(_N_MIN)

    def body(j, v):
        cand = jnp.bitwise_or(v, jnp.left_shift(jnp.int32(1), 30 - j))
        ge = jnp.sum((bits >= cand).astype(jnp.float32))
        return jnp.where(ge >= k, cand, v)

    v = lax.fori_loop(0, 31, body, jnp.int32(0))
    vval = lax.bitcast_convert_type(v, jnp.float32)
    gt = bits > v
    c_gt = jnp.sum(gt.astype(jnp.float32))
    s_gt = jnp.sum(jnp.where(gt, x, 0.0))
    res = (s_gt + (k - c_gt) * vval) / k
    out_ref[...] = jnp.full((8, 128), res, jnp.float32)


_topk_sum_call = pl.pallas_call(
    _topk_sum_body,
    out_shape=jax.ShapeDtypeStruct((8, 128), jnp.float32),
)


def kernel(logits, labels):
    tabs = _bincount_call()(labels.reshape(-1))
    counts = jnp.sum(tabs.reshape(_NW, _NUM_CLASSES, 16),
                     axis=(0, 2)).astype(jnp.float32)
    w = (1.0 / jnp.log(1.02 + counts / _N_PIX)).astype(jnp.float32)

    psum, pcnt = _loss_stats_call(w, logits, labels)
    sum_gt = jnp.sum(psum[:, 0, 0])
    cnt = jnp.sum(pcnt[:, 0, 0])

    def thresh_branch(_):
        return sum_gt / cnt

    def topk_branch(_):
        loss = _loss_values_call(w, logits, labels)
        return _topk_sum_call(loss.reshape(2048, 1024))[0, 0]

    return lax.cond(cnt > _N_MIN, thresh_branch, topk_branch, None)


# speedup vs baseline: 1.0263x; 1.0263x over previous
import functools
import math

import jax
import jax.numpy as jnp
from jax import lax
from jax.experimental import pallas as pl
from jax.experimental.pallas import tpu as pltpu
from jax.experimental.pallas import tpu_sc as plsc

_NUM_CLASSES = 19
_THRESH = -math.log(0.7)
_N_MIN = 131072
_N_PIX = 8 * 512 * 512
_R = 64
_GB = 512 // _R
_GRID = 8 * _GB

_NW = 32
_PER_W = _N_PIX // _NW
_UNROLL = 8
_VREGS_PER_W = _PER_W // 16


def _bincount_body(lbl_hbm, out_hbm, lbl_v, tab_v):
    wid = lax.axis_index("s") * 2 + lax.axis_index("c")
    base = wid * _PER_W
    for c in range(_NUM_CLASSES):
        tab_v[pl.ds(c * 16, 16)] = jnp.zeros((16,), jnp.int32)
    pltpu.sync_copy(lbl_hbm.at[pl.ds(base, _PER_W)], lbl_v)
    lane = lax.iota(jnp.int32, 16)
    ones = jnp.ones((16,), jnp.int32)

    def body(i, carry):
        for u in range(_UNROLL):
            v = lbl_v[pl.ds((i * _UNROLL + u) * 16, 16)]
            plsc.addupdate_scatter(tab_v, [v * 16 + lane], ones)
        return carry

    lax.fori_loop(0, _VREGS_PER_W // _UNROLL, body, 0)
    pltpu.sync_copy(tab_v, out_hbm.at[wid])


@functools.lru_cache(maxsize=None)
def _bincount_call():
    return pl.kernel(
        _bincount_body,
        mesh=plsc.VectorSubcoreMesh(core_axis_name="c", subcore_axis_name="s"),
        out_type=jax.ShapeDtypeStruct((_NW, _NUM_CLASSES * 16), jnp.int32),
        scratch_types=[
            pltpu.VMEM((_PER_W,), jnp.int32),
            pltpu.VMEM((_NUM_CLASSES * 16,), jnp.int32),
        ],
        compiler_params=pltpu.CompilerParams(needs_layout_passes=False),
    )


def _loss_core(w_ref, x_ref, lbl_ref):
    x = x_ref[0]
    lbl = lbl_ref[0]
    m = x[0]
    for c in range(1, _NUM_CLASSES):
        m = jnp.maximum(m, x[c])
    s = jnp.exp(x[0] - m)
    for c in range(1, _NUM_CLASSES):
        s = s + jnp.exp(x[c] - m)
    lse = jnp.log(s) + m
    acc_x = x[0]
    acc_w = jnp.full(m.shape, w_ref[0], jnp.float32)
    for c in range(1, _NUM_CLASSES):
        sel = lbl == c
        acc_x = jnp.where(sel, x[c], acc_x)
        acc_w = jnp.where(sel, w_ref[c], acc_w)
    return acc_w * (lse - acc_x)


def _loss_stats_body(w_ref, x_ref, lbl_ref, sum_ref, cnt_ref):
    loss = _loss_core(w_ref, x_ref, lbl_ref)
    mask = loss > _THRESH
    sum_ref[0] = jnp.full((8, 128), jnp.sum(jnp.where(mask, loss, 0.0)),
                          jnp.float32)
    cnt_ref[0] = jnp.full((8, 128), jnp.sum(mask.astype(jnp.float32)),
                          jnp.float32)


def _loss_values_body(w_ref, x_ref, lbl_ref, loss_ref):
    loss_ref[0] = _loss_core(w_ref, x_ref, lbl_ref)


_dense_in_specs = [
    pl.BlockSpec(memory_space=pltpu.SMEM),
    pl.BlockSpec((1, _NUM_CLASSES, _R, 512), lambda i: (i // _GB, 0, i % _GB, 0)),
    pl.BlockSpec((1, _R, 512), lambda i: (i // _GB, i % _GB, 0)),
]

_loss_stats_call = pl.pallas_call(
    _loss_stats_body,
    grid=(_GRID,),
    in_specs=_dense_in_specs,
    out_specs=[
        pl.BlockSpec((1, 8, 128), lambda i: (i, 0, 0)),
        pl.BlockSpec((1, 8, 128), lambda i: (i, 0, 0)),
    ],
    out_shape=[
        jax.ShapeDtypeStruct((_GRID, 8, 128), jnp.float32),
        jax.ShapeDtypeStruct((_GRID, 8, 128), jnp.float32),
    ],
    compiler_params=pltpu.CompilerParams(dimension_semantics=("parallel",)),
)

_loss_values_call = pl.pallas_call(
    _loss_values_body,
    grid=(_GRID,),
    in_specs=_dense_in_specs,
    out_specs=pl.BlockSpec((1, _R, 512), lambda i: (i // _GB, i % _GB, 0)),
    out_shape=jax.ShapeDtypeStruct((8, 512, 512), jnp.float32),
    compiler_params=pltpu.CompilerParams(dimension_semantics=("parallel",)),
)


def _topk_sum_body(x_ref, out_ref):
    x = x_ref[...]
    bits = lax.bitcast_convert_type(x, jnp.int32)
    k = jnp.float32(_N_MIN)

    def body(j, v):
        cand = jnp.bitwise_or(v, jnp.left_shift(jnp.int32(1), 30 - j))
        ge = jnp.sum((bits >= cand).astype(jnp.float32))
        return jnp.where(ge >= k, cand, v)

    v = lax.fori_loop(0, 31, body, jnp.int32(0))
    vval = lax.bitcast_convert_type(v, jnp.float32)
    gt = bits > v
    c_gt = jnp.sum(gt.astype(jnp.float32))
    s_gt = jnp.sum(jnp.where(gt, x, 0.0))
    res = (s_gt + (k - c_gt) * vval) / k
    out_ref[...] = jnp.full((8, 128), res, jnp.float32)


_topk_sum_call = pl.pallas_call(
    _topk_sum_body,
    out_shape=jax.ShapeDtypeStruct((8, 128), jnp.float32),
)


def kernel(logits, labels):
    tabs = _bincount_call()(labels.reshape(-1))
    counts = jnp.sum(tabs.reshape(_NW, _NUM_CLASSES, 16),
                     axis=(0, 2)).astype(jnp.float32)
    w = (1.0 / jnp.log(1.02 + counts / _N_PIX)).astype(jnp.float32)

    psum, pcnt = _loss_stats_call(w, logits, labels)
    sum_gt = jnp.sum(psum[:, 0, 0])
    cnt = jnp.sum(pcnt[:, 0, 0])

    def thresh_branch(_):
        return sum_gt / cnt

    def topk_branch(_):
        loss = _loss_values_call(w, logits, labels)
        return _topk_sum_call(loss.reshape(2048, 1024))[0, 0]

    return lax.cond(cnt > _N_MIN, thresh_branch, topk_branch, None)


# speedup vs baseline: 1.4470x; 1.4099x over previous
import functools
import math

import jax
import jax.numpy as jnp
from jax import lax
from jax.experimental import pallas as pl
from jax.experimental.pallas import tpu as pltpu
from jax.experimental.pallas import tpu_sc as plsc

_NUM_CLASSES = 19
_THRESH = -math.log(0.7)
_N_MIN = 131072
_N_PIX = 8 * 512 * 512
_R = 128
_GB = 512 // _R
_GRID = 8 * _GB

_NW = 32
_ROWS_W = 4096 // _NW


def _bincount_body(lbl_hbm, out_hbm, lbl_v, tab_v):
    wid = lax.axis_index("s") * 2 + lax.axis_index("c")
    b = wid // 4
    r0 = (wid % 4) * _ROWS_W
    for c in range(_NUM_CLASSES):
        tab_v[pl.ds(c * 16, 16)] = jnp.zeros((16,), jnp.int32)
    pltpu.sync_copy(lbl_hbm.at[b, pl.ds(r0, _ROWS_W)], lbl_v)
    lane = lax.iota(jnp.int32, 16)
    ones = jnp.ones((16,), jnp.int32)

    def body(i, carry):
        for u in range(32):
            v = lbl_v[i, pl.ds(u * 16, 16)]
            plsc.addupdate_scatter(tab_v, [v * 16 + lane], ones)
        return carry

    lax.fori_loop(0, _ROWS_W, body, 0)
    pltpu.sync_copy(tab_v, out_hbm.at[wid])


@functools.lru_cache(maxsize=None)
def _bincount_call():
    return pl.kernel(
        _bincount_body,
        mesh=plsc.VectorSubcoreMesh(core_axis_name="c", subcore_axis_name="s"),
        out_type=jax.ShapeDtypeStruct((_NW, _NUM_CLASSES * 16), jnp.int32),
        scratch_types=[
            pltpu.VMEM((_ROWS_W, 512), jnp.int32),
            pltpu.VMEM((_NUM_CLASSES * 16,), jnp.int32),
        ],
        compiler_params=pltpu.CompilerParams(needs_layout_passes=False),
    )


def _nll_body(x_ref, lbl_ref, nll_ref):
    x = x_ref[0]
    lbl = lbl_ref[0]
    m = x[0]
    for c in range(1, _NUM_CLASSES):
        m = jnp.maximum(m, x[c])
    s = jnp.exp(x[0] - m)
    for c in range(1, _NUM_CLASSES):
        s = s + jnp.exp(x[c] - m)
    lse = jnp.log(s) + m
    acc_x = x[0]
    for c in range(1, _NUM_CLASSES):
        acc_x = jnp.where(lbl == c, x[c], acc_x)
    nll_ref[0] = lse - acc_x


_nll_call = pl.pallas_call(
    _nll_body,
    grid=(_GRID,),
    in_specs=[
        pl.BlockSpec((1, _NUM_CLASSES, _R, 512),
                     lambda i: (i // _GB, 0, i % _GB, 0)),
        pl.BlockSpec((1, _R, 512), lambda i: (i // _GB, i % _GB, 0)),
    ],
    out_specs=pl.BlockSpec((1, _R, 512), lambda i: (i // _GB, i % _GB, 0)),
    out_shape=jax.ShapeDtypeStruct((8, 512, 512), jnp.float32),
    compiler_params=pltpu.CompilerParams(dimension_semantics=("parallel",)),
)


def _wsel(w_ref, lbl):
    acc_w = jnp.full(lbl.shape, w_ref[0], jnp.float32)
    for c in range(1, _NUM_CLASSES):
        acc_w = jnp.where(lbl == c, w_ref[c], acc_w)
    return acc_w


def _stats_body(w_ref, nll_ref, lbl_ref, sum_ref, cnt_ref):
    loss = _wsel(w_ref, lbl_ref[0]) * nll_ref[0]
    mask = loss > _THRESH
    sum_ref[0] = jnp.full((8, 128), jnp.sum(jnp.where(mask, loss, 0.0)),
                          jnp.float32)
    cnt_ref[0] = jnp.full((8, 128), jnp.sum(mask.astype(jnp.float32)),
                          jnp.float32)


_stats_call = pl.pallas_call(
    _stats_body,
    grid=(_GRID,),
    in_specs=[
        pl.BlockSpec(memory_space=pltpu.SMEM),
        pl.BlockSpec((1, _R, 512), lambda i: (i // _GB, i % _GB, 0)),
        pl.BlockSpec((1, _R, 512), lambda i: (i // _GB, i % _GB, 0)),
    ],
    out_specs=[
        pl.BlockSpec((1, 8, 128), lambda i: (i, 0, 0)),
        pl.BlockSpec((1, 8, 128), lambda i: (i, 0, 0)),
    ],
    out_shape=[
        jax.ShapeDtypeStruct((_GRID, 8, 128), jnp.float32),
        jax.ShapeDtypeStruct((_GRID, 8, 128), jnp.float32),
    ],
    compiler_params=pltpu.CompilerParams(dimension_semantics=("parallel",)),
)


def _topk_sum_body(w_ref, nll_ref, lbl_ref, out_ref):
    x = _wsel(w_ref, lbl_ref[...]) * nll_ref[...]
    bits = lax.bitcast_convert_type(x, jnp.int32)
    k = jnp.float32(_N_MIN)

    def body(j, v):
        cand = jnp.bitwise_or(v, jnp.left_shift(jnp.int32(1), 30 - j))
        ge = jnp.sum((bits >= cand).astype(jnp.float32))
        return jnp.where(ge >= k, cand, v)

    v = lax.fori_loop(0, 31, body, jnp.int32(0))
    vval = lax.bitcast_convert_type(v, jnp.float32)
    gt = bits > v
    c_gt = jnp.sum(gt.astype(jnp.float32))
    s_gt = jnp.sum(jnp.where(gt, x, 0.0))
    res = (s_gt + (k - c_gt) * vval) / k
    out_ref[...] = jnp.full((8, 128), res, jnp.float32)


_topk_sum_call = pl.pallas_call(
    _topk_sum_body,
    in_specs=[
        pl.BlockSpec(memory_space=pltpu.SMEM),
        pl.BlockSpec((2048, 1024), lambda: (0, 0)),
        pl.BlockSpec((2048, 1024), lambda: (0, 0)),
    ],
    out_shape=jax.ShapeDtypeStruct((8, 128), jnp.float32),
)


def kernel(logits, labels):
    tabs = _bincount_call()(labels)
    nll = _nll_call(logits, labels)
    counts = jnp.sum(tabs.reshape(_NW, _NUM_CLASSES, 16),
                     axis=(0, 2)).astype(jnp.float32)
    w = (1.0 / jnp.log(1.02 + counts / _N_PIX)).astype(jnp.float32)

    psum, pcnt = _stats_call(w, nll, labels)
    sum_gt = jnp.sum(psum[:, 0, 0])
    cnt = jnp.sum(pcnt[:, 0, 0])

    def thresh_branch(_):
        return sum_gt / cnt

    def topk_branch(_):
        return _topk_sum_call(w, nll.reshape(2048, 1024),
                              labels.reshape(2048, 1024))[0, 0]

    return lax.cond(cnt > _N_MIN, thresh_branch, topk_branch, None)


# speedup vs baseline: 1.5418x; 1.0655x over previous
import functools
import math

import jax
import jax.numpy as jnp
from jax import lax
from jax.experimental import pallas as pl
from jax.experimental.pallas import tpu as pltpu
from jax.experimental.pallas import tpu_sc as plsc

_NUM_CLASSES = 19
_THRESH = -math.log(0.7)
_N_MIN = 131072
_N_PIX = 8 * 512 * 512
_R = 128
_GB = 512 // _R
_GRID = 8 * _GB

_NW = 32
_ROWS_W = 4096 // _NW


def _bincount_body(lbl_hbm, out_hbm, lbl_v, tab_v):
    wid = lax.axis_index("s") * 2 + lax.axis_index("c")
    b = wid // 4
    r0 = (wid % 4) * _ROWS_W
    for c in range(_NUM_CLASSES):
        tab_v[pl.ds(c * 16, 16)] = jnp.zeros((16,), jnp.int32)
    pltpu.sync_copy(lbl_hbm.at[b, pl.ds(r0, _ROWS_W)], lbl_v)
    lane = lax.iota(jnp.int32, 16)
    ones = jnp.ones((16,), jnp.int32)

    def body(i, carry):
        for u in range(32):
            v = lbl_v[i, pl.ds(u * 16, 16)]
            plsc.addupdate_scatter(tab_v, [v * 16 + lane], ones)
        return carry

    lax.fori_loop(0, _ROWS_W, body, 0)
    pltpu.sync_copy(tab_v, out_hbm.at[wid])


@functools.lru_cache(maxsize=None)
def _bincount_call():
    return pl.kernel(
        _bincount_body,
        mesh=plsc.VectorSubcoreMesh(core_axis_name="c", subcore_axis_name="s"),
        out_type=jax.ShapeDtypeStruct((_NW, _NUM_CLASSES * 16), jnp.int32),
        scratch_types=[
            pltpu.VMEM((_ROWS_W, 512), jnp.int32),
            pltpu.VMEM((_NUM_CLASSES * 16,), jnp.int32),
        ],
        compiler_params=pltpu.CompilerParams(needs_layout_passes=False),
    )


def _nll_body(x_ref, lbl_ref, nll_ref):
    x = x_ref[0]
    lbl = lbl_ref[0]
    m = x[0]
    for c in range(1, _NUM_CLASSES):
        m = jnp.maximum(m, x[c])
    s = jnp.exp(x[0] - m)
    for c in range(1, _NUM_CLASSES):
        s = s + jnp.exp(x[c] - m)
    lse = jnp.log(s) + m
    acc_x = x[0]
    for c in range(1, _NUM_CLASSES):
        acc_x = jnp.where(lbl == c, x[c], acc_x)
    nll_ref[0] = lse - acc_x


_nll_call = pl.pallas_call(
    _nll_body,
    grid=(_GRID,),
    in_specs=[
        pl.BlockSpec((1, _NUM_CLASSES, _R, 512),
                     lambda i: (i // _GB, 0, i % _GB, 0)),
        pl.BlockSpec((1, _R, 512), lambda i: (i // _GB, i % _GB, 0)),
    ],
    out_specs=pl.BlockSpec((1, _R, 512), lambda i: (i // _GB, i % _GB, 0)),
    out_shape=jax.ShapeDtypeStruct((8, 512, 512), jnp.float32),
    compiler_params=pltpu.CompilerParams(dimension_semantics=("parallel",)),
)


def _wsel(w_ref, lbl):
    acc_w = jnp.full(lbl.shape, w_ref[0], jnp.float32)
    for c in range(1, _NUM_CLASSES):
        acc_w = jnp.where(lbl == c, w_ref[c], acc_w)
    return acc_w


def _stats_body(w_ref, nll_ref, lbl_ref, sum_ref, cnt_ref):
    loss = _wsel(w_ref, lbl_ref[0]) * nll_ref[0]
    mask = loss > _THRESH
    sum_ref[0] = jnp.full((8, 128), jnp.sum(jnp.where(mask, loss, 0.0)),
                          jnp.float32)
    cnt_ref[0] = jnp.full((8, 128), jnp.sum(mask.astype(jnp.float32)),
                          jnp.float32)


_stats_call = pl.pallas_call(
    _stats_body,
    grid=(8,),
    in_specs=[
        pl.BlockSpec(memory_space=pltpu.SMEM),
        pl.BlockSpec((1, 512, 512), lambda i: (i, 0, 0)),
        pl.BlockSpec((1, 512, 512), lambda i: (i, 0, 0)),
    ],
    out_specs=[
        pl.BlockSpec((1, 8, 128), lambda i: (i, 0, 0)),
        pl.BlockSpec((1, 8, 128), lambda i: (i, 0, 0)),
    ],
    out_shape=[
        jax.ShapeDtypeStruct((8, 8, 128), jnp.float32),
        jax.ShapeDtypeStruct((8, 8, 128), jnp.float32),
    ],
    compiler_params=pltpu.CompilerParams(dimension_semantics=("parallel",)),
)


def _topk_sum_body(w_ref, nll_ref, lbl_ref, out_ref):
    x = _wsel(w_ref, lbl_ref[...]) * nll_ref[...]
    bits = lax.bitcast_convert_type(x, jnp.int32)
    k = jnp.float32(_N_MIN)

    def body(j, v):
        cand = jnp.bitwise_or(v, jnp.left_shift(jnp.int32(1), 30 - j))
        ge = jnp.sum((bits >= cand).astype(jnp.float32))
        return jnp.where(ge >= k, cand, v)

    v = lax.fori_loop(0, 31, body, jnp.int32(0))
    vval = lax.bitcast_convert_type(v, jnp.float32)
    gt = bits > v
    c_gt = jnp.sum(gt.astype(jnp.float32))
    s_gt = jnp.sum(jnp.where(gt, x, 0.0))
    res = (s_gt + (k - c_gt) * vval) / k
    out_ref[...] = jnp.full((8, 128), res, jnp.float32)


_topk_sum_call = pl.pallas_call(
    _topk_sum_body,
    in_specs=[
        pl.BlockSpec(memory_space=pltpu.SMEM),
        pl.BlockSpec((2048, 1024), lambda: (0, 0)),
        pl.BlockSpec((2048, 1024), lambda: (0, 0)),
    ],
    out_shape=jax.ShapeDtypeStruct((8, 128), jnp.float32),
)


def kernel(logits, labels):
    tabs = _bincount_call()(labels)
    nll = _nll_call(logits, labels)
    counts = jnp.sum(tabs.reshape(_NW, _NUM_CLASSES, 16),
                     axis=(0, 2)).astype(jnp.float32)
    w = (1.0 / jnp.log(1.02 + counts / _N_PIX)).astype(jnp.float32)

    psum, pcnt = _stats_call(w, nll, labels)
    sum_gt = jnp.sum(psum[:, 0, 0])
    cnt = jnp.sum(pcnt[:, 0, 0])

    def thresh_branch(_):
        return sum_gt / cnt

    def topk_branch(_):
        return _topk_sum_call(w, nll.reshape(2048, 1024),
                              labels.reshape(2048, 1024))[0, 0]

    return lax.cond(cnt > _N_MIN, thresh_branch, topk_branch, None)


# speedup vs baseline: 1.5482x; 1.0042x over previous
import functools
import math

import jax
import jax.numpy as jnp
from jax import lax
from jax.experimental import pallas as pl
from jax.experimental.pallas import tpu as pltpu
from jax.experimental.pallas import tpu_sc as plsc

_NUM_CLASSES = 19
_THRESH = -math.log(0.7)
_N_MIN = 131072
_N_PIX = 8 * 512 * 512
_R = 128
_GB = 512 // _R
_GRID = 8 * _GB

_NW = 32
_ROWS_W = 4096 // _NW


def _bincount_body(lbl_hbm, out_hbm, lbl_v, tab_v):
    wid = lax.axis_index("s") * 2 + lax.axis_index("c")
    b = wid // 4
    r0 = (wid % 4) * _ROWS_W
    for c in range(_NUM_CLASSES):
        tab_v[pl.ds(c * 16, 16)] = jnp.zeros((16,), jnp.int32)
    pltpu.sync_copy(lbl_hbm.at[b, pl.ds(r0, _ROWS_W)], lbl_v)
    lane = lax.iota(jnp.int32, 16)
    ones = jnp.ones((16,), jnp.int32)

    def body(i, carry):
        for u in range(32):
            v = lbl_v[i, pl.ds(u * 16, 16)]
            plsc.addupdate_scatter(tab_v, [v * 16 + lane], ones)
        return carry

    lax.fori_loop(0, _ROWS_W, body, 0)
    pltpu.sync_copy(tab_v, out_hbm.at[wid])


@functools.lru_cache(maxsize=None)
def _bincount_call():
    return pl.kernel(
        _bincount_body,
        mesh=plsc.VectorSubcoreMesh(core_axis_name="c", subcore_axis_name="s"),
        out_type=jax.ShapeDtypeStruct((_NW, _NUM_CLASSES * 16), jnp.int32),
        scratch_types=[
            pltpu.VMEM((_ROWS_W, 512), jnp.int32),
            pltpu.VMEM((_NUM_CLASSES * 16,), jnp.int32),
        ],
        compiler_params=pltpu.CompilerParams(needs_layout_passes=False),
    )


def _nll_body(x_ref, lbl_ref, nll_ref):
    x = x_ref[0]
    lbl = lbl_ref[0]
    m = x[0]
    for c in range(1, _NUM_CLASSES):
        m = jnp.maximum(m, x[c])
    s = jnp.exp(x[0] - m)
    for c in range(1, _NUM_CLASSES):
        s = s + jnp.exp(x[c] - m)
    lse = jnp.log(s) + m
    acc_x = x[0]
    for c in range(1, _NUM_CLASSES):
        acc_x = jnp.where(lbl == c, x[c], acc_x)
    nll_ref[0] = lse - acc_x


_nll_call = pl.pallas_call(
    _nll_body,
    grid=(_GRID,),
    in_specs=[
        pl.BlockSpec((1, _NUM_CLASSES, _R, 512),
                     lambda i: (i // _GB, 0, i % _GB, 0)),
        pl.BlockSpec((1, _R, 512), lambda i: (i // _GB, i % _GB, 0)),
    ],
    out_specs=pl.BlockSpec((1, _R, 512), lambda i: (i // _GB, i % _GB, 0)),
    out_shape=jax.ShapeDtypeStruct((8, 512, 512), jnp.float32),
    compiler_params=pltpu.CompilerParams(dimension_semantics=("parallel",)),
)


def _wsel(w_ref, lbl):
    acc_w = jnp.full(lbl.shape, w_ref[0], jnp.float32)
    for c in range(1, _NUM_CLASSES):
        acc_w = jnp.where(lbl == c, w_ref[c], acc_w)
    return acc_w


def _stats_body(w_ref, nll_ref, lbl_ref, sum_ref, cnt_ref):
    i = pl.program_id(0)

    @pl.when(i == 0)
    def _():
        sum_ref[...] = jnp.zeros((8, 128), jnp.float32)
        cnt_ref[...] = jnp.zeros((8, 128), jnp.float32)

    loss = _wsel(w_ref, lbl_ref[0]) * nll_ref[0]
    mask = loss > _THRESH
    sum_ref[...] += jnp.sum(jnp.where(mask, loss, 0.0))
    cnt_ref[...] += jnp.sum(mask.astype(jnp.float32))


_stats_call = pl.pallas_call(
    _stats_body,
    grid=(8,),
    in_specs=[
        pl.BlockSpec(memory_space=pltpu.SMEM),
        pl.BlockSpec((1, 512, 512), lambda i: (i, 0, 0)),
        pl.BlockSpec((1, 512, 512), lambda i: (i, 0, 0)),
    ],
    out_specs=[
        pl.BlockSpec((8, 128), lambda i: (0, 0)),
        pl.BlockSpec((8, 128), lambda i: (0, 0)),
    ],
    out_shape=[
        jax.ShapeDtypeStruct((8, 128), jnp.float32),
        jax.ShapeDtypeStruct((8, 128), jnp.float32),
    ],
    compiler_params=pltpu.CompilerParams(dimension_semantics=("arbitrary",)),
)


def _topk_sum_body(w_ref, nll_ref, lbl_ref, out_ref):
    x = _wsel(w_ref, lbl_ref[...]) * nll_ref[...]
    bits = lax.bitcast_convert_type(x, jnp.int32)
    k = jnp.float32(_N_MIN)

    def body(j, v):
        cand = jnp.bitwise_or(v, jnp.left_shift(jnp.int32(1), 30 - j))
        ge = jnp.sum((bits >= cand).astype(jnp.float32))
        return jnp.where(ge >= k, cand, v)

    v = lax.fori_loop(0, 31, body, jnp.int32(0))
    vval = lax.bitcast_convert_type(v, jnp.float32)
    gt = bits > v
    c_gt = jnp.sum(gt.astype(jnp.float32))
    s_gt = jnp.sum(jnp.where(gt, x, 0.0))
    res = (s_gt + (k - c_gt) * vval) / k
    out_ref[...] = jnp.full((8, 128), res, jnp.float32)


_topk_sum_call = pl.pallas_call(
    _topk_sum_body,
    in_specs=[
        pl.BlockSpec(memory_space=pltpu.SMEM),
        pl.BlockSpec((2048, 1024), lambda: (0, 0)),
        pl.BlockSpec((2048, 1024), lambda: (0, 0)),
    ],
    out_shape=jax.ShapeDtypeStruct((8, 128), jnp.float32),
)


def kernel(logits, labels):
    tabs = _bincount_call()(labels)
    nll = _nll_call(logits, labels)
    counts = jnp.sum(tabs.reshape(_NW, _NUM_CLASSES, 16),
                     axis=(0, 2)).astype(jnp.float32)
    w = (1.0 / jnp.log(1.02 + counts / _N_PIX)).astype(jnp.float32)

    psum, pcnt = _stats_call(w, nll, labels)
    sum_gt = psum[0, 0]
    cnt = pcnt[0, 0]

    def thresh_branch(_):
        return sum_gt / cnt

    def topk_branch(_):
        return _topk_sum_call(w, nll.reshape(2048, 1024),
                              labels.reshape(2048, 1024))[0, 0]

    return lax.cond(cnt > _N_MIN, thresh_branch, topk_branch, None)


# speedup vs baseline: 1.6662x; 1.0762x over previous
import functools
import math

import jax
import jax.numpy as jnp
from jax import lax
from jax.experimental import pallas as pl
from jax.experimental.pallas import tpu as pltpu
from jax.experimental.pallas import tpu_sc as plsc

_NUM_CLASSES = 19
_THRESH = -math.log(0.7)
_N_MIN = 131072
_N_PIX = 8 * 512 * 512
_R = 256
_GB = 512 // _R
_GRID = 8 * _GB

_NW = 32
_ROWS_W = 4096 // _NW


def _bincount_body(lbl_hbm, out_hbm, lbl_v, tab_v):
    wid = lax.axis_index("s") * 2 + lax.axis_index("c")
    b = wid // 4
    r0 = (wid % 4) * _ROWS_W
    for c in range(_NUM_CLASSES):
        tab_v[pl.ds(c * 16, 16)] = jnp.zeros((16,), jnp.int32)
    pltpu.sync_copy(lbl_hbm.at[b, pl.ds(r0, _ROWS_W)], lbl_v)
    lane = lax.iota(jnp.int32, 16)
    ones = jnp.ones((16,), jnp.int32)

    def body(i, carry):
        for u in range(32):
            v = lbl_v[i, pl.ds(u * 16, 16)]
            plsc.addupdate_scatter(tab_v, [v * 16 + lane], ones)
        return carry

    lax.fori_loop(0, _ROWS_W, body, 0)
    pltpu.sync_copy(tab_v, out_hbm.at[wid])


@functools.lru_cache(maxsize=None)
def _bincount_call():
    return pl.kernel(
        _bincount_body,
        mesh=plsc.VectorSubcoreMesh(core_axis_name="c", subcore_axis_name="s"),
        out_type=jax.ShapeDtypeStruct((_NW, _NUM_CLASSES * 16), jnp.int32),
        scratch_types=[
            pltpu.VMEM((_ROWS_W, 512), jnp.int32),
            pltpu.VMEM((_NUM_CLASSES * 16,), jnp.int32),
        ],
        compiler_params=pltpu.CompilerParams(needs_layout_passes=False),
    )


def _nll_body(x_ref, lbl_ref, nll_ref):
    x = x_ref[0]
    lbl = lbl_ref[0]
    m = x[0]
    for c in range(1, _NUM_CLASSES):
        m = jnp.maximum(m, x[c])
    s = jnp.exp(x[0] - m)
    for c in range(1, _NUM_CLASSES):
        s = s + jnp.exp(x[c] - m)
    lse = jnp.log(s) + m
    acc_x = x[0]
    for c in range(1, _NUM_CLASSES):
        acc_x = jnp.where(lbl == c, x[c], acc_x)
    nll_ref[0] = lse - acc_x


_nll_call = pl.pallas_call(
    _nll_body,
    grid=(_GRID,),
    in_specs=[
        pl.BlockSpec((1, _NUM_CLASSES, _R, 512),
                     lambda i: (i // _GB, 0, i % _GB, 0)),
        pl.BlockSpec((1, _R, 512), lambda i: (i // _GB, i % _GB, 0)),
    ],
    out_specs=pl.BlockSpec((1, _R, 512), lambda i: (i // _GB, i % _GB, 0)),
    out_shape=jax.ShapeDtypeStruct((8, 512, 512), jnp.float32),
    compiler_params=pltpu.CompilerParams(dimension_semantics=("parallel",)),
)


def _wsel(w_ref, lbl):
    acc_w = jnp.full(lbl.shape, w_ref[0], jnp.float32)
    for c in range(1, _NUM_CLASSES):
        acc_w = jnp.where(lbl == c, w_ref[c], acc_w)
    return acc_w


def _stats_body(w_ref, nll_ref, lbl_ref, sum_ref, cnt_ref):
    i = pl.program_id(0)

    @pl.when(i == 0)
    def _():
        sum_ref[...] = jnp.zeros((8, 128), jnp.float32)
        cnt_ref[...] = jnp.zeros((8, 128), jnp.float32)

    loss = _wsel(w_ref, lbl_ref[0]) * nll_ref[0]
    mask = loss > _THRESH
    sum_ref[...] += jnp.sum(jnp.where(mask, loss, 0.0))
    cnt_ref[...] += jnp.sum(mask.astype(jnp.float32))


_stats_call = pl.pallas_call(
    _stats_body,
    grid=(8,),
    in_specs=[
        pl.BlockSpec(memory_space=pltpu.SMEM),
        pl.BlockSpec((1, 512, 512), lambda i: (i, 0, 0)),
        pl.BlockSpec((1, 512, 512), lambda i: (i, 0, 0)),
    ],
    out_specs=[
        pl.BlockSpec((8, 128), lambda i: (0, 0)),
        pl.BlockSpec((8, 128), lambda i: (0, 0)),
    ],
    out_shape=[
        jax.ShapeDtypeStruct((8, 128), jnp.float32),
        jax.ShapeDtypeStruct((8, 128), jnp.float32),
    ],
    compiler_params=pltpu.CompilerParams(dimension_semantics=("arbitrary",)),
)


def _topk_sum_body(w_ref, nll_ref, lbl_ref, out_ref):
    x = _wsel(w_ref, lbl_ref[...]) * nll_ref[...]
    bits = lax.bitcast_convert_type(x, jnp.int32)
    k = jnp.float32(_N_MIN)

    def body(j, v):
        cand = jnp.bitwise_or(v, jnp.left_shift(jnp.int32(1), 30 - j))
        ge = jnp.sum((bits >= cand).astype(jnp.float32))
        return jnp.where(ge >= k, cand, v)

    v = lax.fori_loop(0, 31, body, jnp.int32(0))
    vval = lax.bitcast_convert_type(v, jnp.float32)
    gt = bits > v
    c_gt = jnp.sum(gt.astype(jnp.float32))
    s_gt = jnp.sum(jnp.where(gt, x, 0.0))
    res = (s_gt + (k - c_gt) * vval) / k
    out_ref[...] = jnp.full((8, 128), res, jnp.float32)


_topk_sum_call = pl.pallas_call(
    _topk_sum_body,
    in_specs=[
        pl.BlockSpec(memory_space=pltpu.SMEM),
        pl.BlockSpec((2048, 1024), lambda: (0, 0)),
        pl.BlockSpec((2048, 1024), lambda: (0, 0)),
    ],
    out_shape=jax.ShapeDtypeStruct((8, 128), jnp.float32),
)


def kernel(logits, labels):
    tabs = _bincount_call()(labels)
    nll = _nll_call(logits, labels)
    counts = jnp.sum(tabs.reshape(_NW, _NUM_CLASSES, 16),
                     axis=(0, 2)).astype(jnp.float32)
    w = (1.0 / jnp.log(1.02 + counts / _N_PIX)).astype(jnp.float32)

    psum, pcnt = _stats_call(w, nll, labels)
    sum_gt = psum[0, 0]
    cnt = pcnt[0, 0]

    def thresh_branch(_):
        return sum_gt / cnt

    def topk_branch(_):
        return _topk_sum_call(w, nll.reshape(2048, 1024),
                              labels.reshape(2048, 1024))[0, 0]

    return lax.cond(cnt > _N_MIN, thresh_branch, topk_branch, None)
